# baseline (speedup 1.0000x reference)
import functools

import jax
import jax.numpy as jnp
import numpy as np
from jax import lax
from jax.experimental import pallas as pl
from jax.experimental.pallas import tpu as pltpu

_CH = 262144

_KS0 = 0
_KS1 = 42
_KS2 = _KS0 ^ _KS1 ^ 0x1BD11BDA

_ROTS = ((13, 15, 26, 6), (17, 29, 16, 24))

_IOTA = np.arange(_CH, dtype=np.int32)


def _rotl(x, r):
    return (x << jnp.int32(r)) | lax.shift_right_logical(x, jnp.int32(32 - r))


def _threefry_scale(x1):
    ks = (_KS0, _KS1, _KS2)
    x0 = None
    for i in range(5):
        for r in _ROTS[i % 2]:
            x0 = x1 if x0 is None else x0 + x1
            x1 = x0 ^ _rotl(x1, r)
        a = ks[(i + 1) % 3]
        if a:
            x0 = x0 + jnp.int32(a)
        x1 = x1 + jnp.int32(ks[(i + 2) % 3] + i + 1)
    bits = x0 ^ x1
    return jnp.where(bits >= 0, jnp.float32(2.0), jnp.float32(0.0))


def _mega_body(nfull, tail, iota_ref, tiota_ref, v_hbm, o_hbm,
               vbuf0, vbuf1, obuf0, obuf1, tvbuf, tobuf, insems, outsems):
    vbufs = (vbuf0, vbuf1)
    obufs = (obuf0, obuf1)

    def in_copy(s, off):
        return pltpu.make_async_copy(
            v_hbm.at[pl.ds(off, _CH)], vbufs[s], insems.at[s])

    def out_copy(s, off):
        return pltpu.make_async_copy(
            obufs[s], o_hbm.at[pl.ds(off, _CH)], outsems.at[s])

    def compute(s, i):
        x1 = iota_ref[...] + (i * jnp.int32(_CH) + jnp.int32(_KS1))
        obufs[s][...] = vbufs[s][...] * _threefry_scale(x1)

    in_copy(0, 0).start()
    in_copy(1, _CH).start()
    for s in (0, 1):
        in_copy(s, s * _CH).wait()
        compute(s, jnp.int32(s))
        in_copy(s, (s + 2) * _CH).start()
        out_copy(s, s * _CH).start()

    def pair(p, carry):
        for s in (0, 1):
            i = p * 2 + s
            off = i * _CH
            in_copy(s, off).wait()
            out_copy(s, off).wait()
            compute(s, i)
            in_copy(s, off + 2 * _CH).start()
            out_copy(s, off).start()
        return carry

    npairs = nfull // 2
    lax.fori_loop(1, npairs - 1, pair, jnp.int32(0))

    for s in (0, 1):
        i = nfull - 2 + s
        off = i * _CH
        in_copy(s, off).wait()
        out_copy(s, off).wait()
        compute(s, jnp.int32(i))
        out_copy(s, off).start()

    toff = nfull * _CH
    tin = pltpu.make_async_copy(
        v_hbm.at[pl.ds(toff, tail)], tvbuf, insems.at[0])
    tin.start()
    out_copy(0, (nfull - 2) * _CH).wait()
    out_copy(1, (nfull - 1) * _CH).wait()
    tin.wait()
    tobuf[...] = tvbuf[...] * _threefry_scale(tiota_ref[...])
    tout = pltpu.make_async_copy(
        tobuf, o_hbm.at[pl.ds(toff, tail)], outsems.at[0])
    tout.start()
    tout.wait()


@jax.jit
def _sparse_dropout(values):
    n = values.shape[0]
    nfull = n // _CH
    if nfull % 2:
        nfull -= 1
    tail = n - nfull * _CH
    tail_iota = np.arange(tail, dtype=np.int32) + np.int32(nfull * _CH + _KS1)
    return pl.pallas_call(
        functools.partial(_mega_body, nfull, tail),
        in_specs=[
            pl.BlockSpec(memory_space=pltpu.VMEM),
            pl.BlockSpec(memory_space=pltpu.VMEM),
            pl.BlockSpec(memory_space=pl.ANY),
        ],
        out_specs=pl.BlockSpec(memory_space=pl.ANY),
        out_shape=jax.ShapeDtypeStruct((n,), jnp.float32),
        scratch_shapes=[
            pltpu.VMEM((_CH,), jnp.float32),
            pltpu.VMEM((_CH,), jnp.float32),
            pltpu.VMEM((_CH,), jnp.float32),
            pltpu.VMEM((_CH,), jnp.float32),
            pltpu.VMEM((tail,), jnp.float32),
            pltpu.VMEM((tail,), jnp.float32),
            pltpu.SemaphoreType.DMA((2,)),
            pltpu.SemaphoreType.DMA((2,)),
        ],
    )(_IOTA, tail_iota, values)


def kernel(indices, values):
    del indices
    return _sparse_dropout(values)


# speedup vs baseline: 1.0811x; 1.0811x over previous
import jax
import jax.numpy as jnp
import numpy as np
from jax import lax
from jax.experimental import pallas as pl

_BLK = 244736

_KS0 = 0
_KS1 = 42
_KS2 = _KS0 ^ _KS1 ^ 0x1BD11BDA

_ROTS = ((13, 15, 26, 6), (17, 29, 16, 24))


def _rotl(x, r):
    return (x << jnp.int32(r)) | lax.shift_right_logical(x, jnp.int32(32 - r))


def _threefry_scale(x1):
    ks = (_KS0, _KS1, _KS2)
    x0 = None
    for i in range(5):
        for j, r in enumerate(_ROTS[i % 2]):
            x0 = x1 if x0 is None else x0 + x1
            x1 = x0 ^ _rotl(x1, r)
        a = ks[(i + 1) % 3]
        if a:
            x0 = x0 + jnp.int32(a)
        x1 = x1 + jnp.int32(ks[(i + 2) % 3] + i + 1)
    bits = x0 ^ x1
    return jnp.where(bits >= 0, jnp.float32(2.0), jnp.float32(0.0))


def _body(ramp_ref, v_ref, o_ref):
    o_ref[...] = v_ref[...] * _threefry_scale(ramp_ref[...])


@jax.jit
def _sparse_dropout(values):
    n = values.shape[0]
    grid = pl.cdiv(n, _BLK)
    ramp = np.arange(n, dtype=np.int32) + np.int32(_KS1)
    return pl.pallas_call(
        _body,
        grid=(grid,),
        in_specs=[
            pl.BlockSpec((_BLK,), lambda i: (i,)),
            pl.BlockSpec((_BLK,), lambda i: (i,)),
        ],
        out_specs=pl.BlockSpec((_BLK,), lambda i: (i,)),
        out_shape=jax.ShapeDtypeStruct((n,), jnp.float32),
    )(ramp, values)


def kernel(indices, values):
    del indices
    return _sparse_dropout(values)
